# baseline (speedup 1.0000x reference)
import functools

import jax
import jax.numpy as jnp
from jax import lax
from jax.experimental import pallas as pl
from jax.experimental.pallas import tpu as pltpu
from jax.experimental.pallas import tpu_sc as plsc

F32 = jnp.float32
BF16 = jnp.bfloat16
I32 = jnp.int32
RPAD = 128
TOPK = 2
BT = 256
FT = 512


def _dot_nt(a, b, prec=None):
    return lax.dot_general(a, b, (((1,), (1,)), ((), ())),
                           preferred_element_type=F32, precision=prec)


def _router_body(x_ref, wg_ref, logits_ref, e12_ref, w12_ref):
    x = x_ref[...]
    wg = wg_ref[...]
    logits = _dot_nt(x, wg, prec=lax.Precision.DEFAULT)
    logits_ref[...] = logits
    T, E = logits.shape
    j = lax.broadcasted_iota(I32, (T, E), 1)
    neg = jnp.finfo(F32).min
    m1 = jnp.max(logits, axis=1, keepdims=True)
    i1 = jnp.min(jnp.where(logits == m1, j, E), axis=1, keepdims=True)
    lm = jnp.where(j == i1, neg, logits)
    m2 = jnp.max(lm, axis=1, keepdims=True)
    i2 = jnp.min(jnp.where(lm == m2, j, E), axis=1, keepdims=True)
    d = jnp.exp(m2 - m1)
    w1 = 1.0 / (1.0 + d)
    w2 = d * w1
    e12_ref[...] = jnp.concatenate([i1, i2], axis=1)
    w12_ref[...] = jnp.concatenate([w1, w2], axis=1)


def _router(x, Wg):
    T, _ = x.shape
    E = Wg.shape[0]
    return pl.pallas_call(
        _router_body,
        out_shape=(jax.ShapeDtypeStruct((T, E), F32),
                   jax.ShapeDtypeStruct((T, TOPK), I32),
                   jax.ShapeDtypeStruct((T, TOPK), F32)),
    )(x, Wg)


def _make_scatter_x(T, H, P, NW, TPW):
    H2 = H // 2
    mesh = plsc.VectorSubcoreMesh(core_axis_name="c", subcore_axis_name="s")

    @functools.partial(
        pl.kernel, mesh=mesh,
        out_type=jax.ShapeDtypeStruct((P, H2), I32),
        scratch_types=[
            pltpu.VMEM((TOPK, TPW), I32),
            pltpu.VMEM((TPW, H2), I32),
            pltpu.SemaphoreType.DMA,
        ],
    )
    def scatter_x(x_hbm, posk_hbm, xs_hbm, idx_v, rows_v, sem):
        wid = lax.axis_index("s") * 2 + lax.axis_index("c")
        base = wid * TPW
        pltpu.sync_copy(posk_hbm.at[wid], idx_v)
        pltpu.sync_copy(x_hbm.at[pl.ds(base, TPW)], rows_v)
        pltpu.async_copy(rows_v, xs_hbm.at[idx_v.at[0]], sem).wait()
        pltpu.async_copy(rows_v, xs_hbm.at[idx_v.at[1]], sem).wait()

    return scatter_x


def _make_combine(T, H, P, NW, TPW):
    TPC = TPW // 2
    mesh = plsc.VectorSubcoreMesh(core_axis_name="c", subcore_axis_name="s")

    @functools.partial(
        pl.kernel, mesh=mesh,
        out_type=jax.ShapeDtypeStruct((T, H), F32),
        scratch_types=[
            pltpu.VMEM((2, TOPK * TPC), I32),
            pltpu.VMEM((TOPK * TPC, H), F32),
            pltpu.VMEM((TPC, H), F32),
            pltpu.SemaphoreType.DMA,
        ],
    )
    def combine(ys_hbm, posi_hbm, out_hbm, idx_v, rows_v, acc_v, sem):
        wid = lax.axis_index("s") * 2 + lax.axis_index("c")
        pltpu.sync_copy(posi_hbm.at[wid], idx_v)
        for c in range(2):
            pltpu.async_copy(ys_hbm.at[idx_v.at[c]], rows_v, sem).wait()

            def tok(i, carry):
                def lane(q, carry2):
                    sl = pl.ds(q * 16, 16)
                    acc_v[i, sl] = rows_v[2 * i, sl] + rows_v[2 * i + 1, sl]
                    return carry2
                return lax.fori_loop(0, H // 16, lane, carry)

            lax.fori_loop(0, TPC, tok, 0)
            pltpu.sync_copy(acc_v, out_hbm.at[pl.ds(wid * TPW + c * TPC, TPC)])

    return combine


def _gemm_body(s_ref, xs_ref, rw_ref, w1_ref, w3_ref, w2_ref,
               u1_ref, v1_ref, u3_ref, v3_ref, v2_ref, u2_ref,
               out_ref, ys_s, xv1_s, xv3_s, hv2_s, *, nf):
    f = pl.program_id(0)
    g = pl.program_id(1)
    xb = xs_ref[...]

    @pl.when(f == 0)
    def _():
        xv1_s[g] = _dot_nt(xb, v1_ref[0]).astype(BF16)
        xv3_s[g] = _dot_nt(xb, v3_ref[0]).astype(BF16)
        hv2_s[g] = jnp.zeros_like(hv2_s[g])

    gate = _dot_nt(xb, w1_ref[0]) + _dot_nt(xv1_s[g], u1_ref[0])
    up = _dot_nt(xb, w3_ref[0]) + _dot_nt(xv3_s[g], u3_ref[0])
    h = (gate * (1.0 / (1.0 + jnp.exp(-gate))) * up).astype(BF16)
    part = _dot_nt(h, w2_ref[0])
    hv2_s[g] += _dot_nt(h, v2_ref[0])

    @pl.when(f == 0)
    def _():
        ys_s[g] = part

    @pl.when((f > 0) & (f < nf - 1))
    def _():
        ys_s[g] += part

    @pl.when(f == nf - 1)
    def _():
        ylr = _dot_nt(hv2_s[g].astype(BF16), u2_ref[0])
        w = rw_ref[0, 0, :]
        out_ref[...] = (ys_s[g] + part + ylr) * w[:, None]


def _grouped_gemm(blk_e, xs, rw3, W1, W3, W2, U1, V1, U3, V3, V2, U2, G):
    P, H = xs.shape
    E, FF, _ = W1.shape
    nf = FF // FT
    grid = (nf, G)
    specs = [
        pl.BlockSpec((BT, H), lambda f, g, s: (g, 0)),
        pl.BlockSpec((1, 1, BT), lambda f, g, s: (g, 0, 0)),
        pl.BlockSpec((1, FT, H), lambda f, g, s: (s[g], f, 0)),
        pl.BlockSpec((1, FT, H), lambda f, g, s: (s[g], f, 0)),
        pl.BlockSpec((1, H, FT), lambda f, g, s: (s[g], 0, f)),
        pl.BlockSpec((1, FT, RPAD), lambda f, g, s: (s[g], f, 0)),
        pl.BlockSpec((1, RPAD, H), lambda f, g, s: (s[g], 0, 0)),
        pl.BlockSpec((1, FT, RPAD), lambda f, g, s: (s[g], f, 0)),
        pl.BlockSpec((1, RPAD, H), lambda f, g, s: (s[g], 0, 0)),
        pl.BlockSpec((1, RPAD, FT), lambda f, g, s: (s[g], 0, f)),
        pl.BlockSpec((1, H, RPAD), lambda f, g, s: (s[g], 0, 0)),
    ]
    grid_spec = pltpu.PrefetchScalarGridSpec(
        num_scalar_prefetch=1,
        grid=grid,
        in_specs=specs,
        out_specs=pl.BlockSpec(
            (BT, H), lambda f, g, s: (jnp.where(f == nf - 1, g, 0), 0)),
        scratch_shapes=[
            pltpu.VMEM((G, BT, H), F32),
            pltpu.VMEM((G, BT, RPAD), BF16),
            pltpu.VMEM((G, BT, RPAD), BF16),
            pltpu.VMEM((G, BT, RPAD), F32),
        ],
    )
    return pl.pallas_call(
        functools.partial(_gemm_body, nf=nf),
        grid_spec=grid_spec,
        out_shape=jax.ShapeDtypeStruct((P, H), F32),
        compiler_params=pltpu.CompilerParams(
            dimension_semantics=("arbitrary", "arbitrary")),
    )(blk_e, xs, rw3, W1, W3, W2, U1, V1, U3, V3, V2, U2)


def _padr(a, axis):
    pad = [(0, 0)] * a.ndim
    pad[axis] = (0, RPAD - a.shape[axis])
    return jnp.pad(a, pad)


def kernel(hidden_states, Wg, W1, W2, W3, U1, V1, U2, V2, U3, V3):
    b, s, h = hidden_states.shape
    x = hidden_states.reshape(-1, h)
    T, H = x.shape
    E, FF, _ = W1.shape
    A = T * TOPK
    G = A // BT + E
    P = G * BT
    NW = 32
    TPW = T // NW

    logits, e12, w12 = _router(x, Wg)

    ea = e12.reshape(-1)
    onehot = (ea[:, None] == jnp.arange(E, dtype=I32)).astype(I32)
    csum = jnp.cumsum(onehot, axis=0) - onehot
    rank = jnp.take_along_axis(csum, ea[:, None], axis=1)[:, 0]
    counts = jnp.sum(onehot, axis=0)
    padc = ((counts + BT - 1) // BT) * BT
    ends = jnp.cumsum(padc)
    pado = ends - padc
    pos = (pado[ea] + rank).astype(I32)
    blk_e = jnp.clip(
        jnp.sum(jnp.arange(G, dtype=I32)[:, None] * BT >= ends[None, :],
                axis=1), 0, E - 1).astype(I32)
    rw = jnp.zeros((P,), F32).at[pos].set(w12.reshape(-1))
    rw3 = rw.reshape(G, 1, BT)
    posk = pos.reshape(NW, TPW, TOPK).transpose(0, 2, 1)
    posi = pos.reshape(NW, 2, TPW)

    xi = lax.bitcast_convert_type(
        x.astype(BF16).reshape(T, H // 2, 2), I32)
    xsi = _make_scatter_x(T, H, P, NW, TPW)(xi, posk)
    xs = lax.bitcast_convert_type(xsi, BF16).reshape(P, H)
    ys = _grouped_gemm(
        blk_e, xs, rw3,
        W1.astype(BF16), W3.astype(BF16), W2.astype(BF16),
        _padr(U1, 2).astype(BF16), _padr(V1, 1).astype(BF16),
        _padr(U3, 2).astype(BF16), _padr(V3, 1).astype(BF16),
        _padr(V2, 1).astype(BF16), _padr(U2, 2).astype(BF16), G)
    out = _make_combine(T, H, P, NW, TPW)(ys, posi)
    return (out.reshape(b, s, h), logits)


# speedup vs baseline: 1.4276x; 1.4276x over previous
import functools

import jax
import jax.numpy as jnp
from jax import lax
from jax.experimental import pallas as pl
from jax.experimental.pallas import tpu as pltpu
from jax.experimental.pallas import tpu_sc as plsc

F32 = jnp.float32
BF16 = jnp.bfloat16
I32 = jnp.int32
RPAD = 128
TOPK = 2
BT = 256
FT = 512


def _dot_nt(a, b, prec=None):
    return lax.dot_general(a, b, (((1,), (1,)), ((), ())),
                           preferred_element_type=F32, precision=prec)


def _router_body(x_ref, wg_ref, logits_ref, e12_ref, w12_ref):
    x = x_ref[...]
    wg = wg_ref[...]
    logits = _dot_nt(x, wg, prec=lax.Precision.DEFAULT)
    logits_ref[...] = logits
    T, E = logits.shape
    j = lax.broadcasted_iota(I32, (T, E), 1)
    neg = jnp.finfo(F32).min
    m1 = jnp.max(logits, axis=1, keepdims=True)
    i1 = jnp.min(jnp.where(logits == m1, j, E), axis=1, keepdims=True)
    lm = jnp.where(j == i1, neg, logits)
    m2 = jnp.max(lm, axis=1, keepdims=True)
    i2 = jnp.min(jnp.where(lm == m2, j, E), axis=1, keepdims=True)
    d = jnp.exp(m2 - m1)
    w1 = 1.0 / (1.0 + d)
    w2 = d * w1
    e12_ref[...] = jnp.concatenate([i1, i2], axis=1)
    w12_ref[...] = jnp.concatenate([w1, w2], axis=1)


def _router(x, Wg):
    T, _ = x.shape
    E = Wg.shape[0]
    return pl.pallas_call(
        _router_body,
        out_shape=(jax.ShapeDtypeStruct((T, E), F32),
                   jax.ShapeDtypeStruct((T, TOPK), I32),
                   jax.ShapeDtypeStruct((T, TOPK), F32)),
    )(x, Wg)


def _make_scatter_x(T, H, P, NW, TPW):
    mesh = plsc.VectorSubcoreMesh(core_axis_name="c", subcore_axis_name="s")

    @functools.partial(
        pl.kernel, mesh=mesh,
        out_type=jax.ShapeDtypeStruct((P, H), F32),
        scratch_types=[
            pltpu.VMEM((TOPK, TPW), I32),
            pltpu.VMEM((TPW, H), F32),
            pltpu.SemaphoreType.DMA,
        ],
    )
    def scatter_x(x_hbm, posk_hbm, xs_hbm, idx_v, rows_v, sem):
        wid = lax.axis_index("s") * 2 + lax.axis_index("c")
        base = wid * TPW
        pltpu.sync_copy(posk_hbm.at[wid], idx_v)
        pltpu.sync_copy(x_hbm.at[pl.ds(base, TPW)], rows_v)
        pltpu.async_copy(rows_v, xs_hbm.at[idx_v.at[0]], sem).wait()
        pltpu.async_copy(rows_v, xs_hbm.at[idx_v.at[1]], sem).wait()

    return scatter_x


def _make_combine(T, H, P, NW, TPW):
    TPC = TPW // 2
    mesh = plsc.VectorSubcoreMesh(core_axis_name="c", subcore_axis_name="s")

    @functools.partial(
        pl.kernel, mesh=mesh,
        out_type=jax.ShapeDtypeStruct((T, H), F32),
        scratch_types=[
            pltpu.VMEM((2, TOPK * TPC), I32),
            pltpu.VMEM((TOPK * TPC, H), F32),
            pltpu.VMEM((TPC, H), F32),
            pltpu.SemaphoreType.DMA,
        ],
    )
    def combine(ys_hbm, posi_hbm, out_hbm, idx_v, rows_v, acc_v, sem):
        wid = lax.axis_index("s") * 2 + lax.axis_index("c")
        pltpu.sync_copy(posi_hbm.at[wid], idx_v)
        for c in range(2):
            pltpu.async_copy(ys_hbm.at[idx_v.at[c]], rows_v, sem).wait()

            def tok(i, carry):
                def lane(q, carry2):
                    sl = pl.ds(q * 16, 16)
                    acc_v[i, sl] = rows_v[2 * i, sl] + rows_v[2 * i + 1, sl]
                    return carry2
                return lax.fori_loop(0, H // 16, lane, carry)

            lax.fori_loop(0, TPC, tok, 0)
            pltpu.sync_copy(acc_v, out_hbm.at[pl.ds(wid * TPW + c * TPC, TPC)])

    return combine


def _gemm_body(s_ref, xs_ref, rw_ref, w1_ref, w3_ref, w2_ref,
               u1_ref, v1_ref, u3_ref, v3_ref, v2_ref, u2_ref,
               out_ref, ys_s, xv1_s, xv3_s, hv2_s, *, nf):
    f = pl.program_id(0)
    g = pl.program_id(1)
    xb = xs_ref[...]

    @pl.when(f == 0)
    def _():
        xv1_s[g] = _dot_nt(xb, v1_ref[0])
        xv3_s[g] = _dot_nt(xb, v3_ref[0])
        hv2_s[g] = jnp.zeros_like(hv2_s[g])

    gate = _dot_nt(xb, w1_ref[0]) + _dot_nt(xv1_s[g], u1_ref[0])
    up = _dot_nt(xb, w3_ref[0]) + _dot_nt(xv3_s[g], u3_ref[0])
    h = gate * (1.0 / (1.0 + jnp.exp(-gate))) * up
    part = _dot_nt(h, w2_ref[0])
    hv2_s[g] += _dot_nt(h, v2_ref[0])

    @pl.when(f == 0)
    def _():
        ys_s[g] = part

    @pl.when((f > 0) & (f < nf - 1))
    def _():
        ys_s[g] += part

    @pl.when(f == nf - 1)
    def _():
        ylr = _dot_nt(hv2_s[g], u2_ref[0])
        w = rw_ref[0, 0, :]
        out_ref[...] = (ys_s[g] + part + ylr) * w[:, None]


def _grouped_gemm(blk_e, xs, rw3, W1, W3, W2, U1, V1, U3, V3, V2, U2, G):
    P, H = xs.shape
    E, FF, _ = W1.shape
    RD = U1.shape[2]
    nf = FF // FT
    grid = (nf, G)
    specs = [
        pl.BlockSpec((BT, H), lambda f, g, s: (g, 0)),
        pl.BlockSpec((1, 1, BT), lambda f, g, s: (g, 0, 0)),
        pl.BlockSpec((1, FT, H), lambda f, g, s: (s[g], f, 0)),
        pl.BlockSpec((1, FT, H), lambda f, g, s: (s[g], f, 0)),
        pl.BlockSpec((1, H, FT), lambda f, g, s: (s[g], 0, f)),
        pl.BlockSpec((1, FT, RD), lambda f, g, s: (s[g], f, 0)),
        pl.BlockSpec((1, RD, H), lambda f, g, s: (s[g], 0, 0)),
        pl.BlockSpec((1, FT, RD), lambda f, g, s: (s[g], f, 0)),
        pl.BlockSpec((1, RD, H), lambda f, g, s: (s[g], 0, 0)),
        pl.BlockSpec((1, RD, FT), lambda f, g, s: (s[g], 0, f)),
        pl.BlockSpec((1, H, RD), lambda f, g, s: (s[g], 0, 0)),
    ]
    grid_spec = pltpu.PrefetchScalarGridSpec(
        num_scalar_prefetch=1,
        grid=grid,
        in_specs=specs,
        out_specs=pl.BlockSpec(
            (BT, H), lambda f, g, s: (jnp.where(f == nf - 1, g, 0), 0)),
        scratch_shapes=[
            pltpu.VMEM((G, BT, H), F32),
            pltpu.VMEM((G, BT, RD), F32),
            pltpu.VMEM((G, BT, RD), F32),
            pltpu.VMEM((G, BT, RD), F32),
        ],
    )
    return pl.pallas_call(
        functools.partial(_gemm_body, nf=nf),
        grid_spec=grid_spec,
        out_shape=jax.ShapeDtypeStruct((P, H), F32),
        compiler_params=pltpu.CompilerParams(
            dimension_semantics=("arbitrary", "arbitrary")),
    )(blk_e, xs, rw3, W1, W3, W2, U1, V1, U3, V3, V2, U2)


def _padr(a, axis):
    pad = [(0, 0)] * a.ndim
    pad[axis] = (0, RPAD - a.shape[axis])
    return jnp.pad(a, pad)


def kernel(hidden_states, Wg, W1, W2, W3, U1, V1, U2, V2, U3, V3):
    b, s, h = hidden_states.shape
    x = hidden_states.reshape(-1, h)
    T, H = x.shape
    E, FF, _ = W1.shape
    A = T * TOPK
    G = A // BT + E
    P = G * BT
    NW = 32
    TPW = T // NW

    logits, e12, w12 = _router(x, Wg)

    ea = e12.reshape(-1)
    onehot = (ea[:, None] == jnp.arange(E, dtype=I32)).astype(I32)
    csum = jnp.cumsum(onehot, axis=0) - onehot
    rank = jnp.take_along_axis(csum, ea[:, None], axis=1)[:, 0]
    counts = jnp.sum(onehot, axis=0)
    padc = ((counts + BT - 1) // BT) * BT
    ends = jnp.cumsum(padc)
    pado = ends - padc
    pos = (pado[ea] + rank).astype(I32)
    blk_e = jnp.clip(
        jnp.sum(jnp.arange(G, dtype=I32)[:, None] * BT >= ends[None, :],
                axis=1), 0, E - 1).astype(I32)
    rw = jnp.zeros((P,), F32).at[pos].set(w12.reshape(-1))
    rw3 = rw.reshape(G, 1, BT)
    posk = pos.reshape(NW, TPW, TOPK).transpose(0, 2, 1)
    posi = pos.reshape(NW, 2, TPW)

    xs = _make_scatter_x(T, H, P, NW, TPW)(x, posk)
    ys = _grouped_gemm(
        blk_e, xs, rw3, W1, W3, W2, U1, V1, U3, V3, V2, U2, G)
    out = _make_combine(T, H, P, NW, TPW)(ys, posi)
    return (out.reshape(b, s, h), logits)


# speedup vs baseline: 1.7096x; 1.1976x over previous
import functools

import jax
import jax.numpy as jnp
from jax import lax
from jax.experimental import pallas as pl
from jax.experimental.pallas import tpu as pltpu
from jax.experimental.pallas import tpu_sc as plsc

F32 = jnp.float32
BF16 = jnp.bfloat16
I32 = jnp.int32
RPAD = 128
TOPK = 2
BT = 256
FT = 1024


def _dot_nt(a, b, prec=None):
    return lax.dot_general(a, b, (((1,), (1,)), ((), ())),
                           preferred_element_type=F32, precision=prec)


def _router_body(x_ref, wg_ref, logits_ref, e12_ref, w12_ref):
    x = x_ref[...]
    wg = wg_ref[...]
    logits = _dot_nt(x, wg, prec=lax.Precision.DEFAULT)
    logits_ref[...] = logits
    T, E = logits.shape
    j = lax.broadcasted_iota(I32, (T, E), 1)
    neg = jnp.finfo(F32).min
    m1 = jnp.max(logits, axis=1, keepdims=True)
    i1 = jnp.min(jnp.where(logits == m1, j, E), axis=1, keepdims=True)
    lm = jnp.where(j == i1, neg, logits)
    m2 = jnp.max(lm, axis=1, keepdims=True)
    i2 = jnp.min(jnp.where(lm == m2, j, E), axis=1, keepdims=True)
    d = jnp.exp(m2 - m1)
    w1 = 1.0 / (1.0 + d)
    w2 = d * w1
    e12_ref[...] = jnp.concatenate([i1, i2], axis=1)
    w12_ref[...] = jnp.concatenate([w1, w2], axis=1)


def _router(x, Wg):
    T, _ = x.shape
    E = Wg.shape[0]
    return pl.pallas_call(
        _router_body,
        out_shape=(jax.ShapeDtypeStruct((T, E), F32),
                   jax.ShapeDtypeStruct((T, TOPK), I32),
                   jax.ShapeDtypeStruct((T, TOPK), F32)),
    )(x, Wg)


def _make_scatter_x(T, H, P, NW, TPW):
    mesh = plsc.VectorSubcoreMesh(core_axis_name="c", subcore_axis_name="s")

    @functools.partial(
        pl.kernel, mesh=mesh,
        out_type=jax.ShapeDtypeStruct((P, H), F32),
        scratch_types=[
            pltpu.VMEM((TOPK, TPW), I32),
            pltpu.VMEM((TPW, H), F32),
            pltpu.SemaphoreType.DMA,
        ],
    )
    def scatter_x(x_hbm, posk_hbm, xs_hbm, idx_v, rows_v, sem):
        wid = lax.axis_index("s") * 2 + lax.axis_index("c")
        base = wid * TPW
        pltpu.sync_copy(posk_hbm.at[wid], idx_v)
        pltpu.sync_copy(x_hbm.at[pl.ds(base, TPW)], rows_v)
        pltpu.async_copy(rows_v, xs_hbm.at[idx_v.at[0]], sem).wait()
        pltpu.async_copy(rows_v, xs_hbm.at[idx_v.at[1]], sem).wait()

    return scatter_x


def _make_combine(T, H, P, NW, TPW):
    TPC = TPW // 2
    mesh = plsc.VectorSubcoreMesh(core_axis_name="c", subcore_axis_name="s")

    @functools.partial(
        pl.kernel, mesh=mesh,
        out_type=jax.ShapeDtypeStruct((T, H), F32),
        scratch_types=[
            pltpu.VMEM((2, TOPK * TPC), I32),
            pltpu.VMEM((TOPK * TPC, H), F32),
            pltpu.VMEM((TPC, H), F32),
            pltpu.SemaphoreType.DMA,
        ],
    )
    def combine(ys_hbm, posi_hbm, out_hbm, idx_v, rows_v, acc_v, sem):
        wid = lax.axis_index("s") * 2 + lax.axis_index("c")
        pltpu.sync_copy(posi_hbm.at[wid], idx_v)
        for c in range(2):
            pltpu.async_copy(ys_hbm.at[idx_v.at[c]], rows_v, sem).wait()

            def tok(i, carry):
                def lane(q, carry2):
                    sl = pl.ds(q * 16, 16)
                    acc_v[i, sl] = rows_v[2 * i, sl] + rows_v[2 * i + 1, sl]
                    return carry2
                return lax.fori_loop(0, H // 16, lane, carry)

            lax.fori_loop(0, TPC, tok, 0)
            pltpu.sync_copy(acc_v, out_hbm.at[pl.ds(wid * TPW + c * TPC, TPC)])

    return combine


def _gemm_body(s_ref, xs_ref, rw_ref, w1_ref, w3_ref, w2_ref,
               u1_ref, v1_ref, u3_ref, v3_ref, v2_ref, u2_ref,
               out_ref, ys_s, xv1_s, xv3_s, hv2_s, *, nf):
    f = pl.program_id(0)
    g = pl.program_id(1)
    xb = xs_ref[...]

    @pl.when(f == 0)
    def _():
        xv1_s[g] = _dot_nt(xb, v1_ref[0])
        xv3_s[g] = _dot_nt(xb, v3_ref[0])
        hv2_s[g] = jnp.zeros_like(hv2_s[g])

    gate = _dot_nt(xb, w1_ref[0]) + _dot_nt(xv1_s[g], u1_ref[0])
    up = _dot_nt(xb, w3_ref[0]) + _dot_nt(xv3_s[g], u3_ref[0])
    h = gate * (1.0 / (1.0 + jnp.exp(-gate))) * up
    part = _dot_nt(h, w2_ref[0])
    hv2_s[g] += _dot_nt(h, v2_ref[0])

    @pl.when(f == 0)
    def _():
        ys_s[g] = part.astype(ys_s.dtype)

    @pl.when((f > 0) & (f < nf - 1))
    def _():
        ys_s[g] = (ys_s[g] + part).astype(ys_s.dtype)

    @pl.when(f == nf - 1)
    def _():
        ylr = _dot_nt(hv2_s[g], u2_ref[0])
        w = rw_ref[0, 0, :]
        out_ref[...] = (ys_s[g] + part + ylr) * w[:, None]


def _grouped_gemm(blk_e, xs, rw3, W1, W3, W2, U1, V1, U3, V3, V2, U2, G):
    P, H = xs.shape
    E, FF, _ = W1.shape
    RD = U1.shape[2]
    nf = FF // FT
    grid = (nf, G)
    specs = [
        pl.BlockSpec((BT, H), lambda f, g, s: (g, 0)),
        pl.BlockSpec((1, 1, BT), lambda f, g, s: (g, 0, 0)),
        pl.BlockSpec((1, FT, H), lambda f, g, s: (s[g], f, 0)),
        pl.BlockSpec((1, FT, H), lambda f, g, s: (s[g], f, 0)),
        pl.BlockSpec((1, H, FT), lambda f, g, s: (s[g], 0, f)),
        pl.BlockSpec((1, FT, RD), lambda f, g, s: (s[g], f, 0)),
        pl.BlockSpec((1, RD, H), lambda f, g, s: (s[g], 0, 0)),
        pl.BlockSpec((1, FT, RD), lambda f, g, s: (s[g], f, 0)),
        pl.BlockSpec((1, RD, H), lambda f, g, s: (s[g], 0, 0)),
        pl.BlockSpec((1, RD, FT), lambda f, g, s: (s[g], 0, f)),
        pl.BlockSpec((1, H, RD), lambda f, g, s: (s[g], 0, 0)),
    ]
    grid_spec = pltpu.PrefetchScalarGridSpec(
        num_scalar_prefetch=1,
        grid=grid,
        in_specs=specs,
        out_specs=pl.BlockSpec(
            (BT, H), lambda f, g, s: (jnp.where(f == nf - 1, g, 0), 0)),
        scratch_shapes=[
            pltpu.VMEM((G, BT, H), BF16),
            pltpu.VMEM((G, BT, RD), F32),
            pltpu.VMEM((G, BT, RD), F32),
            pltpu.VMEM((G, BT, RD), F32),
        ],
    )
    return pl.pallas_call(
        functools.partial(_gemm_body, nf=nf),
        grid_spec=grid_spec,
        out_shape=jax.ShapeDtypeStruct((P, H), F32),
        compiler_params=pltpu.CompilerParams(
            dimension_semantics=("arbitrary", "arbitrary")),
    )(blk_e, xs, rw3, W1, W3, W2, U1, V1, U3, V3, V2, U2)


def _padr(a, axis):
    pad = [(0, 0)] * a.ndim
    pad[axis] = (0, RPAD - a.shape[axis])
    return jnp.pad(a, pad)


def kernel(hidden_states, Wg, W1, W2, W3, U1, V1, U2, V2, U3, V3):
    b, s, h = hidden_states.shape
    x = hidden_states.reshape(-1, h)
    T, H = x.shape
    E, FF, _ = W1.shape
    A = T * TOPK
    G = A // BT + E
    P = G * BT
    NW = 32
    TPW = T // NW

    logits, e12, w12 = _router(x, Wg)

    ea = e12.reshape(-1)
    onehot = (ea[:, None] == jnp.arange(E, dtype=I32)).astype(I32)
    csum = jnp.cumsum(onehot, axis=0) - onehot
    rank = jnp.take_along_axis(csum, ea[:, None], axis=1)[:, 0]
    counts = jnp.sum(onehot, axis=0)
    padc = ((counts + BT - 1) // BT) * BT
    ends = jnp.cumsum(padc)
    pado = ends - padc
    pos = (pado[ea] + rank).astype(I32)
    blk_e = jnp.clip(
        jnp.sum(jnp.arange(G, dtype=I32)[:, None] * BT >= ends[None, :],
                axis=1), 0, E - 1).astype(I32)
    rw = jnp.zeros((P,), F32).at[pos].set(w12.reshape(-1))
    rw3 = rw.reshape(G, 1, BT)
    posk = pos.reshape(NW, TPW, TOPK).transpose(0, 2, 1)
    posi = pos.reshape(NW, 2, TPW)

    xs = _make_scatter_x(T, H, P, NW, TPW)(x, posk)
    ys = _grouped_gemm(
        blk_e, xs, rw3, W1, W3, W2, U1, V1, U3, V3, V2, U2, G)
    out = _make_combine(T, H, P, NW, TPW)(ys, posi)
    return (out.reshape(b, s, h), logits)


# speedup vs baseline: 1.7441x; 1.0202x over previous
import functools

import jax
import jax.numpy as jnp
from jax import lax
from jax.experimental import pallas as pl
from jax.experimental.pallas import tpu as pltpu
from jax.experimental.pallas import tpu_sc as plsc

F32 = jnp.float32
BF16 = jnp.bfloat16
I32 = jnp.int32
RPAD = 128
TOPK = 2
BT = 256
FT = 1024


def _dot_nt(a, b, prec=None):
    return lax.dot_general(a, b, (((1,), (1,)), ((), ())),
                           preferred_element_type=F32, precision=prec)


def _router_body(x_ref, wg_ref, logits_ref, e12_ref, w12_ref):
    x = x_ref[...]
    wg = wg_ref[...]
    logits = _dot_nt(x, wg, prec=lax.Precision.DEFAULT)
    logits_ref[...] = logits
    T, E = logits.shape
    j = lax.broadcasted_iota(I32, (T, E), 1)
    neg = jnp.finfo(F32).min
    m1 = jnp.max(logits, axis=1, keepdims=True)
    i1 = jnp.min(jnp.where(logits == m1, j, E), axis=1, keepdims=True)
    lm = jnp.where(j == i1, neg, logits)
    m2 = jnp.max(lm, axis=1, keepdims=True)
    i2 = jnp.min(jnp.where(lm == m2, j, E), axis=1, keepdims=True)
    d = jnp.exp(m2 - m1)
    w1 = 1.0 / (1.0 + d)
    w2 = d * w1
    e12_ref[...] = jnp.concatenate([i1, i2], axis=1)
    w12_ref[...] = jnp.concatenate([w1, w2], axis=1)


def _router(x, Wg):
    T, _ = x.shape
    E = Wg.shape[0]
    return pl.pallas_call(
        _router_body,
        out_shape=(jax.ShapeDtypeStruct((T, E), F32),
                   jax.ShapeDtypeStruct((T, TOPK), I32),
                   jax.ShapeDtypeStruct((T, TOPK), F32)),
    )(x, Wg)


def _make_scatter_x(T, H, P, NW, TPW):
    mesh = plsc.VectorSubcoreMesh(core_axis_name="c", subcore_axis_name="s")

    @functools.partial(
        pl.kernel, mesh=mesh,
        out_type=jax.ShapeDtypeStruct((P, H), F32),
        scratch_types=[
            pltpu.VMEM((TOPK, TPW), I32),
            pltpu.VMEM((TPW, H), F32),
            pltpu.SemaphoreType.DMA,
        ],
    )
    def scatter_x(x_hbm, posk_hbm, xs_hbm, idx_v, rows_v, sem):
        wid = lax.axis_index("s") * 2 + lax.axis_index("c")
        base = wid * TPW
        pltpu.sync_copy(posk_hbm.at[wid], idx_v)
        pltpu.sync_copy(x_hbm.at[pl.ds(base, TPW)], rows_v)
        pltpu.async_copy(rows_v, xs_hbm.at[idx_v.at[0]], sem).wait()
        pltpu.async_copy(rows_v, xs_hbm.at[idx_v.at[1]], sem).wait()

    return scatter_x


def _make_combine(T, H, P, NW, TPW):
    TPC = TPW // 2
    mesh = plsc.VectorSubcoreMesh(core_axis_name="c", subcore_axis_name="s")

    @functools.partial(
        pl.kernel, mesh=mesh,
        out_type=jax.ShapeDtypeStruct((T, H), F32),
        scratch_types=[
            pltpu.VMEM((2, TOPK * TPC), I32),
            pltpu.VMEM((TOPK * TPC, H), F32),
            pltpu.VMEM((TPC, H), F32),
            pltpu.SemaphoreType.DMA,
        ],
    )
    def combine(ys_hbm, posi_hbm, out_hbm, idx_v, rows_v, acc_v, sem):
        wid = lax.axis_index("s") * 2 + lax.axis_index("c")
        pltpu.sync_copy(posi_hbm.at[wid], idx_v)
        for c in range(2):
            pltpu.async_copy(ys_hbm.at[idx_v.at[c]], rows_v, sem).wait()

            def tok(i, carry):
                def lane(q, carry2):
                    sl = pl.ds(q * 16, 16)
                    acc_v[i, sl] = rows_v[2 * i, sl] + rows_v[2 * i + 1, sl]
                    return carry2
                return lax.fori_loop(0, H // 16, lane, carry)

            lax.fori_loop(0, TPC, tok, 0)
            pltpu.sync_copy(acc_v, out_hbm.at[pl.ds(wid * TPW + c * TPC, TPC)])

    return combine


def _gemm_body(s_ref, xs_ref, rw_ref, w1_ref, w3_ref, w2_ref,
               u1_ref, v1_ref, u3_ref, v3_ref, v2_ref, u2_ref,
               out_ref, ys_s, xv1_s, xv3_s, hv2_s, *, nf):
    f = pl.program_id(0)
    g = pl.program_id(1)
    xb = xs_ref[...]

    @pl.when(f == 0)
    def _():
        xv1_s[g] = _dot_nt(xb, v1_ref[0])
        xv3_s[g] = _dot_nt(xb, v3_ref[0])
        hv2_s[g] = jnp.zeros_like(hv2_s[g])

    gate = _dot_nt(xb, w1_ref[0]) + _dot_nt(xv1_s[g], u1_ref[0])
    up = _dot_nt(xb, w3_ref[0]) + _dot_nt(xv3_s[g], u3_ref[0])
    h = gate * (1.0 / (1.0 + jnp.exp(-gate))) * up
    part = _dot_nt(h, w2_ref[0])
    hv2_s[g] += _dot_nt(h, v2_ref[0])

    @pl.when(f == 0)
    def _():
        ys_s[g] = part.astype(ys_s.dtype)

    @pl.when((f > 0) & (f < nf - 1))
    def _():
        ys_s[g] = (ys_s[g] + part).astype(ys_s.dtype)

    @pl.when(f == nf - 1)
    def _():
        ylr = _dot_nt(hv2_s[g], u2_ref[0])
        w = rw_ref[0, 0, :]
        out_ref[...] = (ys_s[g] + part + ylr) * w[:, None]


def _grouped_gemm(blk_e, xs, rw3, W1, W3, W2, U1, V1, U3, V3, V2, U2, G):
    P, H = xs.shape
    E, FF, _ = W1.shape
    RD = U1.shape[2]
    nf = FF // FT
    grid = (nf, G)
    specs = [
        pl.BlockSpec((BT, H), lambda f, g, s: (g, 0)),
        pl.BlockSpec((1, 1, BT), lambda f, g, s: (g, 0, 0)),
        pl.BlockSpec((1, FT, H), lambda f, g, s: (s[g], f, 0)),
        pl.BlockSpec((1, FT, H), lambda f, g, s: (s[g], f, 0)),
        pl.BlockSpec((1, H, FT), lambda f, g, s: (s[g], 0, f)),
        pl.BlockSpec((1, FT, RD), lambda f, g, s: (s[g], f, 0)),
        pl.BlockSpec((1, RD, H), lambda f, g, s: (s[g], 0, 0)),
        pl.BlockSpec((1, FT, RD), lambda f, g, s: (s[g], f, 0)),
        pl.BlockSpec((1, RD, H), lambda f, g, s: (s[g], 0, 0)),
        pl.BlockSpec((1, RD, FT), lambda f, g, s: (s[g], 0, f)),
        pl.BlockSpec((1, H, RD), lambda f, g, s: (s[g], 0, 0)),
    ]
    grid_spec = pltpu.PrefetchScalarGridSpec(
        num_scalar_prefetch=1,
        grid=grid,
        in_specs=specs,
        out_specs=pl.BlockSpec(
            (BT, H), lambda f, g, s: (jnp.where(f == nf - 1, g, 0), 0)),
        scratch_shapes=[
            pltpu.VMEM((G, BT, H), BF16),
            pltpu.VMEM((G, BT, RD), F32),
            pltpu.VMEM((G, BT, RD), F32),
            pltpu.VMEM((G, BT, RD), F32),
        ],
    )
    return pl.pallas_call(
        functools.partial(_gemm_body, nf=nf),
        grid_spec=grid_spec,
        out_shape=jax.ShapeDtypeStruct((P, H), F32),
        compiler_params=pltpu.CompilerParams(
            dimension_semantics=("arbitrary", "arbitrary")),
    )(blk_e, xs, rw3, W1, W3, W2, U1, V1, U3, V3, V2, U2)


def _padr(a, axis):
    pad = [(0, 0)] * a.ndim
    pad[axis] = (0, RPAD - a.shape[axis])
    return jnp.pad(a, pad)


def kernel(hidden_states, Wg, W1, W2, W3, U1, V1, U2, V2, U3, V3):
    b, s, h = hidden_states.shape
    x = hidden_states.reshape(-1, h)
    T, H = x.shape
    E, FF, _ = W1.shape
    A = T * TOPK
    G = A // BT + E
    P = G * BT
    NW = 32
    TPW = T // NW

    logits, e12, w12 = _router(x, Wg)

    ea = e12.reshape(-1)
    onehot = (ea[:, None] == jnp.arange(E, dtype=I32)).astype(I32)
    csum = jnp.cumsum(onehot, axis=0) - onehot
    rank = jnp.sum(csum * onehot, axis=1)
    counts = jnp.sum(onehot, axis=0)
    padc = ((counts + BT - 1) // BT) * BT
    ends = jnp.cumsum(padc)
    pado = ends - padc
    pos = (jnp.sum(pado[None, :] * onehot, axis=1) + rank).astype(I32)
    blk_e = jnp.clip(
        jnp.sum(jnp.arange(G, dtype=I32)[:, None] * BT >= ends[None, :],
                axis=1), 0, E - 1).astype(I32)
    rw = jnp.zeros((P,), F32).at[pos].set(w12.reshape(-1))
    rw3 = rw.reshape(G, 1, BT)
    posk = pos.reshape(NW, TPW, TOPK).transpose(0, 2, 1)
    posi = pos.reshape(NW, 2, TPW)

    xs = _make_scatter_x(T, H, P, NW, TPW)(x, posk)
    ys = _grouped_gemm(
        blk_e, xs, rw3, W1, W3, W2, U1, V1, U3, V3, V2, U2, G)
    out = _make_combine(T, H, P, NW, TPW)(ys, posi)
    return (out.reshape(b, s, h), logits)
